# Initial kernel scaffold; baseline (speedup 1.0000x reference)
import jax
import jax.numpy as jnp
from jax import lax
from jax.experimental import pallas as pl
from jax.experimental.pallas import tpu as pltpu
from jax.experimental.pallas import tpu_sc as plsc

B = 1024
S = 200
N = B * S
EMB = 64
POS_DIM = 32
OUT_D = EMB + POS_DIM

NW = 32
NT = N // NW
G = 128
C = 640
GPB = C // G
NB = NT // C
NROWS = N // G


def _emb_body(widx_hbm, pidx_hbm, wtab_hbm, ptab_hbm, out_hbm,
              idx_w, idx_p, word_v, pos_v, sem):
    wid = lax.axis_index("s") * 2 + lax.axis_index("c")
    row0_t = wid * (NT // G)
    base_t = wid * NT

    def block(i, _):
        row0 = row0_t + i * GPB
        base = base_t + i * C
        pltpu.sync_copy(widx_hbm.at[pl.ds(row0, GPB)], idx_w)
        pltpu.sync_copy(pidx_hbm.at[pl.ds(row0, GPB)], idx_p)
        copies = []
        for j in range(GPB):
            cw = pltpu.make_async_copy(
                wtab_hbm.at[idx_w.at[j]], word_v.at[pl.ds(j * G, G)], sem)
            cp = pltpu.make_async_copy(
                ptab_hbm.at[idx_p.at[j]], pos_v.at[pl.ds(j * G, G)], sem)
            cw.start()
            cp.start()
            copies.append(cw)
            copies.append(cp)
        for c in copies:
            c.wait()
        pltpu.sync_copy(word_v, out_hbm.at[pl.ds(base, C), pl.ds(0, EMB)])
        pltpu.sync_copy(pos_v, out_hbm.at[pl.ds(base, C), pl.ds(EMB, POS_DIM)])
        return ()

    lax.fori_loop(0, NB, block, (), unroll=False)


@jax.jit
def _emb_call(widx, pidx, word_table, pos_table):
    mesh = plsc.VectorSubcoreMesh(core_axis_name="c", subcore_axis_name="s")
    f = pl.kernel(
        _emb_body,
        out_type=jax.ShapeDtypeStruct((N, OUT_D), jnp.float32),
        mesh=mesh,
        scratch_types=[
            pltpu.VMEM((GPB, G), jnp.int32),
            pltpu.VMEM((GPB, G), jnp.int32),
            pltpu.VMEM((C, EMB), jnp.float32),
            pltpu.VMEM((C, POS_DIM), jnp.float32),
            pltpu.SemaphoreType.DMA,
        ],
    )
    return f(widx, pidx, word_table, pos_table)


def kernel(word_inputs, pos_inputs, word_table, pos_table):
    widx = word_inputs.astype(jnp.int32).reshape(NROWS, G)
    pidx = pos_inputs.astype(jnp.int32).reshape(NROWS, G)
    out = _emb_call(widx, pidx, word_table, pos_table)
    return out.reshape(B, S, OUT_D)


# speedup vs baseline: 1.0557x; 1.0557x over previous
import jax
import jax.numpy as jnp
from jax import lax
from jax.experimental import pallas as pl
from jax.experimental.pallas import tpu as pltpu
from jax.experimental.pallas import tpu_sc as plsc

B = 1024
S = 200
N = B * S
EMB = 64
POS_DIM = 32
OUT_D = EMB + POS_DIM

NW = 32
NT = N // NW
G = 128
C = 640
GPB = C // G
NB = NT // C
NROWS = N // G


def _emb_body(widx_hbm, pidx_hbm, wtab_hbm, ptab_hbm, out_hbm,
              idx_w, idx_p, word_v, pos_v, sem):
    wid = lax.axis_index("s") * 2 + lax.axis_index("c")
    base_t = wid * NT

    def block(i, _):
        base = base_t + i * C
        pltpu.sync_copy(widx_hbm.at[pl.ds(base, C)], idx_w)
        pltpu.sync_copy(pidx_hbm.at[pl.ds(base, C)], idx_p)
        copies = []
        for j in range(GPB):
            cw = pltpu.make_async_copy(
                wtab_hbm.at[idx_w.at[pl.ds(j * G, G)]],
                word_v.at[pl.ds(j * G, G)], sem)
            cp = pltpu.make_async_copy(
                ptab_hbm.at[idx_p.at[pl.ds(j * G, G)]],
                pos_v.at[pl.ds(j * G, G)], sem)
            cw.start()
            cp.start()
            copies.append(cw)
            copies.append(cp)
        for c in copies:
            c.wait()
        pltpu.sync_copy(word_v, out_hbm.at[pl.ds(base, C), pl.ds(0, EMB)])
        pltpu.sync_copy(pos_v, out_hbm.at[pl.ds(base, C), pl.ds(EMB, POS_DIM)])
        return ()

    lax.fori_loop(0, NB, block, (), unroll=False)


@jax.jit
def _emb_call(widx, pidx, word_table, pos_table):
    mesh = plsc.VectorSubcoreMesh(core_axis_name="c", subcore_axis_name="s")
    f = pl.kernel(
        _emb_body,
        out_type=jax.ShapeDtypeStruct((N, OUT_D), jnp.float32),
        mesh=mesh,
        compiler_params=pltpu.CompilerParams(use_tc_tiling_on_sc=False),
        scratch_types=[
            pltpu.VMEM((C,), jnp.int32),
            pltpu.VMEM((C,), jnp.int32),
            pltpu.VMEM((C, EMB), jnp.float32),
            pltpu.VMEM((C, POS_DIM), jnp.float32),
            pltpu.SemaphoreType.DMA,
        ],
    )
    return f(widx, pidx, word_table, pos_table)


def kernel(word_inputs, pos_inputs, word_table, pos_table):
    widx = word_inputs.astype(jnp.int32).reshape(N)
    pidx = pos_inputs.astype(jnp.int32).reshape(N)
    out = _emb_call(widx, pidx, word_table, pos_table)
    return out.reshape(B, S, OUT_D)


# speedup vs baseline: 1.0575x; 1.0017x over previous
import jax
import jax.numpy as jnp
from jax import lax
from jax.experimental import pallas as pl
from jax.experimental.pallas import tpu as pltpu
from jax.experimental.pallas import tpu_sc as plsc

B = 1024
S = 200
N = B * S
EMB = 64
POS_DIM = 32
OUT_D = EMB + POS_DIM

NW = 32
NT = N // NW
G = 128
C = 640
GPB = C // G
NB = NT // C
NROWS = N // G


def _emb_body(widx_hbm, pidx_hbm, wtab_hbm, ptab_hbm, out_hbm,
              idx_w, idx_p, word_v, pos_v, sem_i, sem_g, sem_o):
    wid = lax.axis_index("s") * 2 + lax.axis_index("c")
    base_t = wid * NT

    def idx_copies(g, b):
        base = base_t + g * C
        return [
            pltpu.make_async_copy(widx_hbm.at[pl.ds(base, C)],
                                  idx_w.at[b], sem_i),
            pltpu.make_async_copy(pidx_hbm.at[pl.ds(base, C)],
                                  idx_p.at[b], sem_i),
        ]

    def gather_copies(b):
        cs = []
        for j in range(GPB):
            cs.append(pltpu.make_async_copy(
                wtab_hbm.at[idx_w.at[b, pl.ds(j * G, G)]],
                word_v.at[b, pl.ds(j * G, G)], sem_g))
            cs.append(pltpu.make_async_copy(
                ptab_hbm.at[idx_p.at[b, pl.ds(j * G, G)]],
                pos_v.at[b, pl.ds(j * G, G)], sem_g))
        return cs

    def out_copies(g, b):
        base = base_t + g * C
        return [
            pltpu.make_async_copy(
                word_v.at[b], out_hbm.at[pl.ds(base, C), pl.ds(0, EMB)],
                sem_o),
            pltpu.make_async_copy(
                pos_v.at[b], out_hbm.at[pl.ds(base, C), pl.ds(EMB, POS_DIM)],
                sem_o),
        ]

    idx_d = [idx_copies(g, g % 2) for g in range(NB)]
    gat_d = [gather_copies(g % 2) for g in range(NB)]
    out_d = [out_copies(g, g % 2) for g in range(NB)]

    for c in idx_d[0] + idx_d[1]:
        c.start()
    for g in range(NB):
        for c in idx_d[g]:
            c.wait()
        if g >= 2:
            for c in out_d[g - 2]:
                c.wait()
        for c in gat_d[g]:
            c.start()
        for c in gat_d[g]:
            c.wait()
        for c in out_d[g]:
            c.start()
        if g + 2 < NB:
            for c in idx_d[g + 2]:
                c.start()
    for g in (NB - 2, NB - 1):
        for c in out_d[g]:
            c.wait()


@jax.jit
def _emb_call(widx, pidx, word_table, pos_table):
    mesh = plsc.VectorSubcoreMesh(core_axis_name="c", subcore_axis_name="s")
    f = pl.kernel(
        _emb_body,
        out_type=jax.ShapeDtypeStruct((N, OUT_D), jnp.float32),
        mesh=mesh,
        compiler_params=pltpu.CompilerParams(use_tc_tiling_on_sc=False),
        scratch_types=[
            pltpu.VMEM((2, C), jnp.int32),
            pltpu.VMEM((2, C), jnp.int32),
            pltpu.VMEM((2, C, EMB), jnp.float32),
            pltpu.VMEM((2, C, POS_DIM), jnp.float32),
            pltpu.SemaphoreType.DMA,
            pltpu.SemaphoreType.DMA,
            pltpu.SemaphoreType.DMA,
        ],
    )
    return f(widx, pidx, word_table, pos_table)


def kernel(word_inputs, pos_inputs, word_table, pos_table):
    widx = word_inputs.astype(jnp.int32).reshape(N)
    pidx = pos_inputs.astype(jnp.int32).reshape(N)
    out = _emb_call(widx, pidx, word_table, pos_table)
    return out.reshape(B, S, OUT_D)
